# baseline (speedup 1.0000x reference)
import functools

import jax
import jax.numpy as jnp
from jax import lax
from jax.experimental import pallas as pl
from jax.experimental.pallas import tpu as pltpu
from jax.experimental.pallas import tpu_sc as plsc

N = 100000
D = 128
G = 2048


_GATE_BLK = 800


def _gate_body(x_ref, w_ref, b_ref, out_ref):
    x = x_ref[...]
    gate = jax.nn.sigmoid(jnp.dot(x, w_ref[...].reshape(D, 1))
                          + b_ref[0, 0])
    out_ref[...] = x * gate


def _gate(node_feats, ws_W, ws_b):
    return pl.pallas_call(
        _gate_body,
        grid=(N // _GATE_BLK,),
        in_specs=[
            pl.BlockSpec((_GATE_BLK, D), lambda i: (i, 0)),
            pl.BlockSpec((1, D), lambda i: (0, 0)),
            pl.BlockSpec((1, 1), lambda i: (0, 0)),
        ],
        out_specs=pl.BlockSpec((_GATE_BLK, D), lambda i: (i, 0)),
        out_shape=jax.ShapeDtypeStruct((100224, D), jnp.float32),
    )(node_feats, ws_W.reshape(1, D), ws_b.reshape(1, 1))


_CHUNK = 128
_NP = 100224
_NCH = _NP // _CHUNK
_NW = 32
_KMAX = -(-_NCH // _NW)
_NSLOT = 4
_DEPTH = 2


def _segsum_body(xw_hbm, seg_hbm, zeros_hbm, out_hbm, acc,
                 xbuf0, xbuf1, xbuf2, xbuf3, idx0, idx1, idx2, idx3,
                 sem0, sem1, sem2, sem3, ssem0, ssem1, ssem2, ssem3):
    cid = lax.axis_index("c")
    sid = lax.axis_index("s")
    wid = sid * 2 + cid

    rows = pl.ds(sid * (G // 16), G // 16)
    pltpu.sync_copy(zeros_hbm.at[rows], acc.at[rows])

    xbufs = (xbuf0, xbuf1, xbuf2, xbuf3)
    idxs = (idx0, idx1, idx2, idx3)
    sems = (sem0, sem1, sem2, sem3)
    ssems = (ssem0, ssem1, ssem2, ssem3)

    def start_in(k):
        c = wid + _NW * k
        slot = k % _NSLOT

        @pl.when(c < _NCH)
        def _():
            base = c * _CHUNK
            pltpu.async_copy(xw_hbm.at[pl.ds(base, _CHUNK)], xbufs[slot],
                             sems[slot])
            pltpu.async_copy(seg_hbm.at[pl.ds(base, _CHUNK)],
                             idxs[slot].at[0], sems[slot])

    def wait_in(k):
        c = wid + _NW * k
        slot = k % _NSLOT

        @pl.when(c < _NCH)
        def _():
            pltpu.make_async_copy(xw_hbm.at[pl.ds(0, _CHUNK)], xbufs[slot],
                                  sems[slot]).wait()
            pltpu.make_async_copy(seg_hbm.at[pl.ds(0, _CHUNK)],
                                  idxs[slot].at[0], sems[slot]).wait()

    def fire_scat(k):
        c = wid + _NW * k
        slot = k % _NSLOT

        @pl.when(c < _NCH)
        def _():
            pltpu.async_copy(xbufs[slot], acc.at[idxs[slot].at[0]],
                             ssems[slot], add=True)

    def wait_scat(k):
        if k < 0:
            return
        c = wid + _NW * k
        slot = k % _NSLOT

        @pl.when(c < _NCH)
        def _():
            pltpu.make_async_copy(xbufs[slot], acc.at[idxs[slot].at[0]],
                                  ssems[slot]).wait()

    for k in range(_DEPTH):
        start_in(k)
    plsc.subcore_barrier()

    for k in range(_KMAX):
        wait_in(k)
        fire_scat(k)
        wait_scat(k + _DEPTH - _NSLOT)
        start_in(k + _DEPTH)
    for k in range(_KMAX - _NSLOT + _DEPTH, _KMAX):
        wait_scat(k)

    plsc.subcore_barrier()
    out_rows = pl.ds(cid * G + sid * (G // 16), G // 16)
    pltpu.sync_copy(acc.at[rows], out_hbm.at[out_rows])


def _segsum(xw, segp, zeros):
    mesh = plsc.VectorSubcoreMesh(core_axis_name="c", subcore_axis_name="s")
    return pl.kernel(
        _segsum_body,
        out_type=jax.ShapeDtypeStruct((2 * G, D), jnp.float32),
        mesh=mesh,
        scratch_types=[
            pltpu.VMEM_SHARED((G + 1, D), jnp.float32),
            pltpu.VMEM((_CHUNK, D), jnp.float32),
            pltpu.VMEM((_CHUNK, D), jnp.float32),
            pltpu.VMEM((_CHUNK, D), jnp.float32),
            pltpu.VMEM((_CHUNK, D), jnp.float32),
            pltpu.VMEM((1, 128), jnp.int32),
            pltpu.VMEM((1, 128), jnp.int32),
            pltpu.VMEM((1, 128), jnp.int32),
            pltpu.VMEM((1, 128), jnp.int32),
            pltpu.SemaphoreType.DMA,
            pltpu.SemaphoreType.DMA,
            pltpu.SemaphoreType.DMA,
            pltpu.SemaphoreType.DMA,
            pltpu.SemaphoreType.DMA,
            pltpu.SemaphoreType.DMA,
            pltpu.SemaphoreType.DMA,
            pltpu.SemaphoreType.DMA,
        ],
    )(xw, segp, zeros)


def _head_body(p_ref, fc1_W, fc1_b, bn1_g, bn1_b, fc2_W, fc2_b, bn2_g, bn2_b,
               fc3_W, fc3_b, bn3_g, bn3_b, out1_W, out1_b, out2_W, out2_b,
               out_ref):
    gf = p_ref[:G, :] + p_ref[G:, :]

    def bn(x, g, b, eps=1e-5):
        mu = jnp.mean(x, axis=0, keepdims=True)
        var = jnp.mean((x - mu) * (x - mu), axis=0, keepdims=True)
        return (x - mu) * lax.rsqrt(var + eps) * g + b

    h = bn(jax.nn.relu(jnp.dot(gf, fc1_W[...]) + fc1_b[...]),
           bn1_g[...], bn1_b[...])
    h = bn(jax.nn.relu(jnp.dot(h, fc2_W[...]) + fc2_b[...]),
           bn2_g[...], bn2_b[...])
    h = bn(jax.nn.relu(jnp.dot(h, fc3_W[...]) + fc3_b[...]),
           bn3_g[...], bn3_b[...])
    h = jax.nn.relu(jnp.dot(h, out1_W[...]) + out1_b[...])
    out_ref[...] = jnp.sum(h * out2_W[...], axis=1, keepdims=True) + out2_b[0, 0]


def _head(partials, args):
    vec = lambda: pl.BlockSpec((1, D), lambda: (0, 0))
    full = lambda: pl.BlockSpec((D, D), lambda: (0, 0))
    return pl.pallas_call(
        _head_body,
        in_specs=[pl.BlockSpec((2 * G, D), lambda: (0, 0)),
                  full(), vec(), vec(), vec(),
                  full(), vec(), vec(), vec(),
                  full(), vec(), vec(), vec(),
                  full(), vec(), vec(), pl.BlockSpec((1, 1), lambda: (0, 0))],
        out_specs=pl.BlockSpec((G, 1), lambda: (0, 0)),
        out_shape=jax.ShapeDtypeStruct((G, 1), jnp.float32),
    )(partials, *args)


def kernel(node_feats, segment_ids, ws_W, ws_b, sh_W, sh_b,
           fc1_W, fc1_b, bn1_g, bn1_b,
           fc2_W, fc2_b, bn2_g, bn2_b,
           fc3_W, fc3_b, bn3_g, bn3_b,
           out1_W, out1_b, out2_W, out2_b):
    seg = segment_ids.astype(jnp.int32)
    segp = jnp.concatenate([seg, jnp.full((_NP - N,), G, jnp.int32)])
    xw = _gate(node_feats, ws_W, ws_b)
    zeros = jnp.zeros((G, D), jnp.float32)
    partials = _segsum(xw, segp, zeros)
    r1 = lambda a: a.reshape(1, D)
    args = (fc1_W, r1(fc1_b), r1(bn1_g), r1(bn1_b),
            fc2_W, r1(fc2_b), r1(bn2_g), r1(bn2_b),
            fc3_W, r1(fc3_b), r1(bn3_g), r1(bn3_b),
            out1_W, r1(out1_b), out2_W.reshape(1, D),
            out2_b.reshape(1, 1))
    return _head(partials, args)


# speedup vs baseline: 1.5568x; 1.5568x over previous
import functools

import jax
import jax.numpy as jnp
from jax import lax
from jax.experimental import pallas as pl
from jax.experimental.pallas import tpu as pltpu
from jax.experimental.pallas import tpu_sc as plsc

N = 100000
D = 128
G = 2048


_GATE_BLK = 4000


def _gate_body(x_ref, w_ref, b_ref, out_ref):
    x = x_ref[...]
    gate = jax.nn.sigmoid(jnp.dot(x, w_ref[...].reshape(D, 1))
                          + b_ref[0, 0])
    out_ref[...] = x * gate


def _gate(node_feats, ws_W, ws_b):
    return pl.pallas_call(
        _gate_body,
        grid=(N // _GATE_BLK,),
        in_specs=[
            pl.BlockSpec((_GATE_BLK, D), lambda i: (i, 0)),
            pl.BlockSpec((1, D), lambda i: (0, 0)),
            pl.BlockSpec((1, 1), lambda i: (0, 0)),
        ],
        out_specs=pl.BlockSpec((_GATE_BLK, D), lambda i: (i, 0)),
        out_shape=jax.ShapeDtypeStruct((100224, D), jnp.float32),
    )(node_feats, ws_W.reshape(1, D), ws_b.reshape(1, 1))


_CHUNK = 128
_NP = 100224
_NCH = _NP // _CHUNK
_NW = 32
_KMAX = -(-_NCH // _NW)
_NSLOT = 4
_DEPTH = 2


def _segsum_body(xw_hbm, seg_hbm, zeros_hbm, out_hbm, acc,
                 xbuf0, xbuf1, xbuf2, xbuf3, idx0, idx1, idx2, idx3,
                 sem0, sem1, sem2, sem3, ssem0, ssem1, ssem2, ssem3):
    cid = lax.axis_index("c")
    sid = lax.axis_index("s")
    wid = sid * 2 + cid

    rows = pl.ds(sid * (G // 16), G // 16)
    pltpu.sync_copy(zeros_hbm.at[rows], acc.at[rows])

    xbufs = (xbuf0, xbuf1, xbuf2, xbuf3)
    idxs = (idx0, idx1, idx2, idx3)
    sems = (sem0, sem1, sem2, sem3)
    ssems = (ssem0, ssem1, ssem2, ssem3)

    def start_in(k):
        c = wid + _NW * k
        slot = k % _NSLOT

        @pl.when(c < _NCH)
        def _():
            base = c * _CHUNK
            pltpu.async_copy(xw_hbm.at[pl.ds(base, _CHUNK)], xbufs[slot],
                             sems[slot])
            pltpu.async_copy(seg_hbm.at[pl.ds(base, _CHUNK)],
                             idxs[slot].at[0], sems[slot])

    def wait_in(k):
        c = wid + _NW * k
        slot = k % _NSLOT

        @pl.when(c < _NCH)
        def _():
            pltpu.make_async_copy(xw_hbm.at[pl.ds(0, _CHUNK)], xbufs[slot],
                                  sems[slot]).wait()
            pltpu.make_async_copy(seg_hbm.at[pl.ds(0, _CHUNK)],
                                  idxs[slot].at[0], sems[slot]).wait()

    def fire_scat(k):
        c = wid + _NW * k
        slot = k % _NSLOT

        @pl.when(c < _NCH)
        def _():
            pltpu.async_copy(xbufs[slot], acc.at[idxs[slot].at[0]],
                             ssems[slot], add=True)

    def wait_scat(k):
        if k < 0:
            return
        c = wid + _NW * k
        slot = k % _NSLOT

        @pl.when(c < _NCH)
        def _():
            pltpu.make_async_copy(xbufs[slot], acc.at[idxs[slot].at[0]],
                                  ssems[slot]).wait()

    for k in range(_DEPTH):
        start_in(k)
    plsc.subcore_barrier()

    for k in range(_KMAX):
        wait_in(k)
        fire_scat(k)
        wait_scat(k + _DEPTH - _NSLOT)
        start_in(k + _DEPTH)
    for k in range(_KMAX - _NSLOT + _DEPTH, _KMAX):
        wait_scat(k)

    plsc.subcore_barrier()
    out_rows = pl.ds(cid * G + sid * (G // 16), G // 16)
    pltpu.sync_copy(acc.at[rows], out_hbm.at[out_rows])


def _segsum(xw, segp, zeros):
    mesh = plsc.VectorSubcoreMesh(core_axis_name="c", subcore_axis_name="s")
    return pl.kernel(
        _segsum_body,
        out_type=jax.ShapeDtypeStruct((2 * G, D), jnp.float32),
        mesh=mesh,
        scratch_types=[
            pltpu.VMEM_SHARED((G + 1, D), jnp.float32),
            pltpu.VMEM((_CHUNK, D), jnp.float32),
            pltpu.VMEM((_CHUNK, D), jnp.float32),
            pltpu.VMEM((_CHUNK, D), jnp.float32),
            pltpu.VMEM((_CHUNK, D), jnp.float32),
            pltpu.VMEM((1, 128), jnp.int32),
            pltpu.VMEM((1, 128), jnp.int32),
            pltpu.VMEM((1, 128), jnp.int32),
            pltpu.VMEM((1, 128), jnp.int32),
            pltpu.SemaphoreType.DMA,
            pltpu.SemaphoreType.DMA,
            pltpu.SemaphoreType.DMA,
            pltpu.SemaphoreType.DMA,
            pltpu.SemaphoreType.DMA,
            pltpu.SemaphoreType.DMA,
            pltpu.SemaphoreType.DMA,
            pltpu.SemaphoreType.DMA,
        ],
    )(xw, segp, zeros)


def _head_body(p_ref, fc1_W, fc1_b, bn1_g, bn1_b, fc2_W, fc2_b, bn2_g, bn2_b,
               fc3_W, fc3_b, bn3_g, bn3_b, out1_W, out1_b, out2_W, out2_b,
               out_ref):
    gf = p_ref[:G, :] + p_ref[G:, :]

    def bn(x, g, b, eps=1e-5):
        mu = jnp.mean(x, axis=0, keepdims=True)
        var = jnp.mean((x - mu) * (x - mu), axis=0, keepdims=True)
        return (x - mu) * lax.rsqrt(var + eps) * g + b

    h = bn(jax.nn.relu(jnp.dot(gf, fc1_W[...]) + fc1_b[...]),
           bn1_g[...], bn1_b[...])
    h = bn(jax.nn.relu(jnp.dot(h, fc2_W[...]) + fc2_b[...]),
           bn2_g[...], bn2_b[...])
    h = bn(jax.nn.relu(jnp.dot(h, fc3_W[...]) + fc3_b[...]),
           bn3_g[...], bn3_b[...])
    h = jax.nn.relu(jnp.dot(h, out1_W[...]) + out1_b[...])
    out_ref[...] = jnp.sum(h * out2_W[...], axis=1, keepdims=True) + out2_b[0, 0]


def _head(partials, args):
    vec = lambda: pl.BlockSpec((1, D), lambda: (0, 0))
    full = lambda: pl.BlockSpec((D, D), lambda: (0, 0))
    return pl.pallas_call(
        _head_body,
        in_specs=[pl.BlockSpec((2 * G, D), lambda: (0, 0)),
                  full(), vec(), vec(), vec(),
                  full(), vec(), vec(), vec(),
                  full(), vec(), vec(), vec(),
                  full(), vec(), vec(), pl.BlockSpec((1, 1), lambda: (0, 0))],
        out_specs=pl.BlockSpec((G, 1), lambda: (0, 0)),
        out_shape=jax.ShapeDtypeStruct((G, 1), jnp.float32),
    )(partials, *args)


def kernel(node_feats, segment_ids, ws_W, ws_b, sh_W, sh_b,
           fc1_W, fc1_b, bn1_g, bn1_b,
           fc2_W, fc2_b, bn2_g, bn2_b,
           fc3_W, fc3_b, bn3_g, bn3_b,
           out1_W, out1_b, out2_W, out2_b):
    seg = segment_ids.astype(jnp.int32)
    segp = jnp.concatenate([seg, jnp.full((_NP - N,), G, jnp.int32)])
    xw = _gate(node_feats, ws_W, ws_b)
    zeros = jnp.zeros((G, D), jnp.float32)
    partials = _segsum(xw, segp, zeros)
    r1 = lambda a: a.reshape(1, D)
    args = (fc1_W, r1(fc1_b), r1(bn1_g), r1(bn1_b),
            fc2_W, r1(fc2_b), r1(bn2_g), r1(bn2_b),
            fc3_W, r1(fc3_b), r1(bn3_g), r1(bn3_b),
            out1_W, r1(out1_b), out2_W.reshape(1, D),
            out2_b.reshape(1, 1))
    return _head(partials, args)


# speedup vs baseline: 1.7027x; 1.0937x over previous
import functools

import jax
import jax.numpy as jnp
from jax import lax
from jax.experimental import pallas as pl
from jax.experimental.pallas import tpu as pltpu
from jax.experimental.pallas import tpu_sc as plsc

N = 100000
D = 128
G = 2048


_GATE_BLK = 10000


def _gate_body(x_ref, w_ref, b_ref, out_ref):
    x = x_ref[...]
    gate = jax.nn.sigmoid(jnp.dot(x, w_ref[...].reshape(D, 1))
                          + b_ref[0, 0])
    out_ref[...] = x * gate


def _gate(node_feats, ws_W, ws_b):
    return pl.pallas_call(
        _gate_body,
        grid=(N // _GATE_BLK,),
        in_specs=[
            pl.BlockSpec((_GATE_BLK, D), lambda i: (i, 0)),
            pl.BlockSpec((1, D), lambda i: (0, 0)),
            pl.BlockSpec((1, 1), lambda i: (0, 0)),
        ],
        out_specs=pl.BlockSpec((_GATE_BLK, D), lambda i: (i, 0)),
        out_shape=jax.ShapeDtypeStruct((100224, D), jnp.float32),
    )(node_feats, ws_W.reshape(1, D), ws_b.reshape(1, 1))


_CHUNK = 128
_NP = 100224
_NCH = _NP // _CHUNK
_NW = 32
_KMAX = -(-_NCH // _NW)
_NSLOT = 4
_DEPTH = 2


def _segsum_body(xw_hbm, seg_hbm, zeros_hbm, out_hbm, acc,
                 xbuf0, xbuf1, xbuf2, xbuf3, idx0, idx1, idx2, idx3,
                 sem0, sem1, sem2, sem3, ssem0, ssem1, ssem2, ssem3):
    cid = lax.axis_index("c")
    sid = lax.axis_index("s")
    wid = sid * 2 + cid

    rows = pl.ds(sid * (G // 16), G // 16)
    pltpu.sync_copy(zeros_hbm.at[rows], acc.at[rows])

    xbufs = (xbuf0, xbuf1, xbuf2, xbuf3)
    idxs = (idx0, idx1, idx2, idx3)
    sems = (sem0, sem1, sem2, sem3)
    ssems = (ssem0, ssem1, ssem2, ssem3)

    def start_in(k):
        c = wid + _NW * k
        slot = k % _NSLOT

        @pl.when(c < _NCH)
        def _():
            base = c * _CHUNK
            pltpu.async_copy(xw_hbm.at[pl.ds(base, _CHUNK)], xbufs[slot],
                             sems[slot])
            pltpu.async_copy(seg_hbm.at[pl.ds(base, _CHUNK)],
                             idxs[slot].at[0], sems[slot])

    def wait_in(k):
        c = wid + _NW * k
        slot = k % _NSLOT

        @pl.when(c < _NCH)
        def _():
            pltpu.make_async_copy(xw_hbm.at[pl.ds(0, _CHUNK)], xbufs[slot],
                                  sems[slot]).wait()
            pltpu.make_async_copy(seg_hbm.at[pl.ds(0, _CHUNK)],
                                  idxs[slot].at[0], sems[slot]).wait()

    def fire_scat(k):
        c = wid + _NW * k
        slot = k % _NSLOT

        @pl.when(c < _NCH)
        def _():
            pltpu.async_copy(xbufs[slot], acc.at[idxs[slot].at[0]],
                             ssems[slot], add=True)

    def wait_scat(k):
        if k < 0:
            return
        c = wid + _NW * k
        slot = k % _NSLOT

        @pl.when(c < _NCH)
        def _():
            pltpu.make_async_copy(xbufs[slot], acc.at[idxs[slot].at[0]],
                                  ssems[slot]).wait()

    for k in range(_DEPTH):
        start_in(k)
    plsc.subcore_barrier()

    for k in range(_KMAX):
        wait_in(k)
        fire_scat(k)
        wait_scat(k + _DEPTH - _NSLOT)
        start_in(k + _DEPTH)
    for k in range(_KMAX - _NSLOT + _DEPTH, _KMAX):
        wait_scat(k)

    plsc.subcore_barrier()
    out_rows = pl.ds(cid * G + sid * (G // 16), G // 16)
    pltpu.sync_copy(acc.at[rows], out_hbm.at[out_rows])


def _segsum(xw, segp, zeros):
    mesh = plsc.VectorSubcoreMesh(core_axis_name="c", subcore_axis_name="s")
    return pl.kernel(
        _segsum_body,
        out_type=jax.ShapeDtypeStruct((2 * G, D), jnp.float32),
        mesh=mesh,
        scratch_types=[
            pltpu.VMEM_SHARED((G + 1, D), jnp.float32),
            pltpu.VMEM((_CHUNK, D), jnp.float32),
            pltpu.VMEM((_CHUNK, D), jnp.float32),
            pltpu.VMEM((_CHUNK, D), jnp.float32),
            pltpu.VMEM((_CHUNK, D), jnp.float32),
            pltpu.VMEM((1, 128), jnp.int32),
            pltpu.VMEM((1, 128), jnp.int32),
            pltpu.VMEM((1, 128), jnp.int32),
            pltpu.VMEM((1, 128), jnp.int32),
            pltpu.SemaphoreType.DMA,
            pltpu.SemaphoreType.DMA,
            pltpu.SemaphoreType.DMA,
            pltpu.SemaphoreType.DMA,
            pltpu.SemaphoreType.DMA,
            pltpu.SemaphoreType.DMA,
            pltpu.SemaphoreType.DMA,
            pltpu.SemaphoreType.DMA,
        ],
    )(xw, segp, zeros)


def _head_body(p_ref, fc1_W, fc1_b, bn1_g, bn1_b, fc2_W, fc2_b, bn2_g, bn2_b,
               fc3_W, fc3_b, bn3_g, bn3_b, out1_W, out1_b, out2_W, out2_b,
               out_ref):
    gf = p_ref[:G, :] + p_ref[G:, :]

    def bn(x, g, b, eps=1e-5):
        mu = jnp.mean(x, axis=0, keepdims=True)
        var = jnp.mean((x - mu) * (x - mu), axis=0, keepdims=True)
        return (x - mu) * lax.rsqrt(var + eps) * g + b

    h = bn(jax.nn.relu(jnp.dot(gf, fc1_W[...]) + fc1_b[...]),
           bn1_g[...], bn1_b[...])
    h = bn(jax.nn.relu(jnp.dot(h, fc2_W[...]) + fc2_b[...]),
           bn2_g[...], bn2_b[...])
    h = bn(jax.nn.relu(jnp.dot(h, fc3_W[...]) + fc3_b[...]),
           bn3_g[...], bn3_b[...])
    h = jax.nn.relu(jnp.dot(h, out1_W[...]) + out1_b[...])
    out_ref[...] = jnp.sum(h * out2_W[...], axis=1, keepdims=True) + out2_b[0, 0]


def _head(partials, args):
    vec = lambda: pl.BlockSpec((1, D), lambda: (0, 0))
    full = lambda: pl.BlockSpec((D, D), lambda: (0, 0))
    return pl.pallas_call(
        _head_body,
        in_specs=[pl.BlockSpec((2 * G, D), lambda: (0, 0)),
                  full(), vec(), vec(), vec(),
                  full(), vec(), vec(), vec(),
                  full(), vec(), vec(), vec(),
                  full(), vec(), vec(), pl.BlockSpec((1, 1), lambda: (0, 0))],
        out_specs=pl.BlockSpec((G, 1), lambda: (0, 0)),
        out_shape=jax.ShapeDtypeStruct((G, 1), jnp.float32),
    )(partials, *args)


def kernel(node_feats, segment_ids, ws_W, ws_b, sh_W, sh_b,
           fc1_W, fc1_b, bn1_g, bn1_b,
           fc2_W, fc2_b, bn2_g, bn2_b,
           fc3_W, fc3_b, bn3_g, bn3_b,
           out1_W, out1_b, out2_W, out2_b):
    seg = segment_ids.astype(jnp.int32)
    segp = jnp.concatenate([seg, jnp.full((_NP - N,), G, jnp.int32)])
    xw = _gate(node_feats, ws_W, ws_b)
    zeros = jnp.zeros((G, D), jnp.float32)
    partials = _segsum(xw, segp, zeros)
    r1 = lambda a: a.reshape(1, D)
    args = (fc1_W, r1(fc1_b), r1(bn1_g), r1(bn1_b),
            fc2_W, r1(fc2_b), r1(bn2_g), r1(bn2_b),
            fc3_W, r1(fc3_b), r1(bn3_g), r1(bn3_b),
            out1_W, r1(out1_b), out2_W.reshape(1, D),
            out2_b.reshape(1, 1))
    return _head(partials, args)


# speedup vs baseline: 1.7068x; 1.0024x over previous
import functools

import jax
import jax.numpy as jnp
from jax import lax
from jax.experimental import pallas as pl
from jax.experimental.pallas import tpu as pltpu
from jax.experimental.pallas import tpu_sc as plsc

N = 100000
D = 128
G = 2048


_GATE_BLK = 20000


def _gate_body(x_ref, w_ref, b_ref, out_ref):
    x = x_ref[...]
    gate = jax.nn.sigmoid(jnp.dot(x, w_ref[...].reshape(D, 1))
                          + b_ref[0, 0])
    out_ref[...] = x * gate


def _gate(node_feats, ws_W, ws_b):
    return pl.pallas_call(
        _gate_body,
        grid=(N // _GATE_BLK,),
        in_specs=[
            pl.BlockSpec((_GATE_BLK, D), lambda i: (i, 0)),
            pl.BlockSpec((1, D), lambda i: (0, 0)),
            pl.BlockSpec((1, 1), lambda i: (0, 0)),
        ],
        out_specs=pl.BlockSpec((_GATE_BLK, D), lambda i: (i, 0)),
        out_shape=jax.ShapeDtypeStruct((100224, D), jnp.float32),
    )(node_feats, ws_W.reshape(1, D), ws_b.reshape(1, 1))


_CHUNK = 128
_NP = 100224
_NCH = _NP // _CHUNK
_NW = 32
_KMAX = -(-_NCH // _NW)
_NSLOT = 4
_DEPTH = 2


def _segsum_body(xw_hbm, seg_hbm, zeros_hbm, out_hbm, acc,
                 xbuf0, xbuf1, xbuf2, xbuf3, idx0, idx1, idx2, idx3,
                 sem0, sem1, sem2, sem3, ssem0, ssem1, ssem2, ssem3):
    cid = lax.axis_index("c")
    sid = lax.axis_index("s")
    wid = sid * 2 + cid

    rows = pl.ds(sid * (G // 16), G // 16)
    pltpu.sync_copy(zeros_hbm.at[rows], acc.at[rows])

    xbufs = (xbuf0, xbuf1, xbuf2, xbuf3)
    idxs = (idx0, idx1, idx2, idx3)
    sems = (sem0, sem1, sem2, sem3)
    ssems = (ssem0, ssem1, ssem2, ssem3)

    def start_in(k):
        c = wid + _NW * k
        slot = k % _NSLOT

        @pl.when(c < _NCH)
        def _():
            base = c * _CHUNK
            pltpu.async_copy(xw_hbm.at[pl.ds(base, _CHUNK)], xbufs[slot],
                             sems[slot])
            pltpu.async_copy(seg_hbm.at[pl.ds(base, _CHUNK)],
                             idxs[slot].at[0], sems[slot])

    def wait_in(k):
        c = wid + _NW * k
        slot = k % _NSLOT

        @pl.when(c < _NCH)
        def _():
            pltpu.make_async_copy(xw_hbm.at[pl.ds(0, _CHUNK)], xbufs[slot],
                                  sems[slot]).wait()
            pltpu.make_async_copy(seg_hbm.at[pl.ds(0, _CHUNK)],
                                  idxs[slot].at[0], sems[slot]).wait()

    def fire_scat(k):
        c = wid + _NW * k
        slot = k % _NSLOT

        @pl.when(c < _NCH)
        def _():
            pltpu.async_copy(xbufs[slot], acc.at[idxs[slot].at[0]],
                             ssems[slot], add=True)

    def wait_scat(k):
        if k < 0:
            return
        c = wid + _NW * k
        slot = k % _NSLOT

        @pl.when(c < _NCH)
        def _():
            pltpu.make_async_copy(xbufs[slot], acc.at[idxs[slot].at[0]],
                                  ssems[slot]).wait()

    for k in range(_DEPTH):
        start_in(k)
    plsc.subcore_barrier()

    for k in range(_KMAX):
        wait_in(k)
        fire_scat(k)
        wait_scat(k + _DEPTH - _NSLOT)
        start_in(k + _DEPTH)
    for k in range(_KMAX - _NSLOT + _DEPTH, _KMAX):
        wait_scat(k)

    plsc.subcore_barrier()
    out_rows = pl.ds(cid * G + sid * (G // 16), G // 16)
    pltpu.sync_copy(acc.at[rows], out_hbm.at[out_rows])


def _segsum(xw, segp, zeros):
    mesh = plsc.VectorSubcoreMesh(core_axis_name="c", subcore_axis_name="s")
    return pl.kernel(
        _segsum_body,
        out_type=jax.ShapeDtypeStruct((2 * G, D), jnp.float32),
        mesh=mesh,
        scratch_types=[
            pltpu.VMEM_SHARED((G + 1, D), jnp.float32),
            pltpu.VMEM((_CHUNK, D), jnp.float32),
            pltpu.VMEM((_CHUNK, D), jnp.float32),
            pltpu.VMEM((_CHUNK, D), jnp.float32),
            pltpu.VMEM((_CHUNK, D), jnp.float32),
            pltpu.VMEM((1, 128), jnp.int32),
            pltpu.VMEM((1, 128), jnp.int32),
            pltpu.VMEM((1, 128), jnp.int32),
            pltpu.VMEM((1, 128), jnp.int32),
            pltpu.SemaphoreType.DMA,
            pltpu.SemaphoreType.DMA,
            pltpu.SemaphoreType.DMA,
            pltpu.SemaphoreType.DMA,
            pltpu.SemaphoreType.DMA,
            pltpu.SemaphoreType.DMA,
            pltpu.SemaphoreType.DMA,
            pltpu.SemaphoreType.DMA,
        ],
    )(xw, segp, zeros)


def _head_body(p_ref, fc1_W, fc1_b, bn1_g, bn1_b, fc2_W, fc2_b, bn2_g, bn2_b,
               fc3_W, fc3_b, bn3_g, bn3_b, out1_W, out1_b, out2_W, out2_b,
               out_ref):
    gf = p_ref[:G, :] + p_ref[G:, :]

    def bn(x, g, b, eps=1e-5):
        mu = jnp.mean(x, axis=0, keepdims=True)
        var = jnp.mean((x - mu) * (x - mu), axis=0, keepdims=True)
        return (x - mu) * lax.rsqrt(var + eps) * g + b

    h = bn(jax.nn.relu(jnp.dot(gf, fc1_W[...]) + fc1_b[...]),
           bn1_g[...], bn1_b[...])
    h = bn(jax.nn.relu(jnp.dot(h, fc2_W[...]) + fc2_b[...]),
           bn2_g[...], bn2_b[...])
    h = bn(jax.nn.relu(jnp.dot(h, fc3_W[...]) + fc3_b[...]),
           bn3_g[...], bn3_b[...])
    h = jax.nn.relu(jnp.dot(h, out1_W[...]) + out1_b[...])
    out_ref[...] = jnp.sum(h * out2_W[...], axis=1, keepdims=True) + out2_b[0, 0]


def _head(partials, args):
    vec = lambda: pl.BlockSpec((1, D), lambda: (0, 0))
    full = lambda: pl.BlockSpec((D, D), lambda: (0, 0))
    return pl.pallas_call(
        _head_body,
        in_specs=[pl.BlockSpec((2 * G, D), lambda: (0, 0)),
                  full(), vec(), vec(), vec(),
                  full(), vec(), vec(), vec(),
                  full(), vec(), vec(), vec(),
                  full(), vec(), vec(), pl.BlockSpec((1, 1), lambda: (0, 0))],
        out_specs=pl.BlockSpec((G, 1), lambda: (0, 0)),
        out_shape=jax.ShapeDtypeStruct((G, 1), jnp.float32),
    )(partials, *args)


def kernel(node_feats, segment_ids, ws_W, ws_b, sh_W, sh_b,
           fc1_W, fc1_b, bn1_g, bn1_b,
           fc2_W, fc2_b, bn2_g, bn2_b,
           fc3_W, fc3_b, bn3_g, bn3_b,
           out1_W, out1_b, out2_W, out2_b):
    seg = segment_ids.astype(jnp.int32)
    segp = jnp.concatenate([seg, jnp.full((_NP - N,), G, jnp.int32)])
    xw = _gate(node_feats, ws_W, ws_b)
    zeros = jnp.zeros((G, D), jnp.float32)
    partials = _segsum(xw, segp, zeros)
    r1 = lambda a: a.reshape(1, D)
    args = (fc1_W, r1(fc1_b), r1(bn1_g), r1(bn1_b),
            fc2_W, r1(fc2_b), r1(bn2_g), r1(bn2_b),
            fc3_W, r1(fc3_b), r1(bn3_g), r1(bn3_b),
            out1_W, r1(out1_b), out2_W.reshape(1, D),
            out2_b.reshape(1, 1))
    return _head(partials, args)
